# Initial kernel scaffold; baseline (speedup 1.0000x reference)
import jax
import jax.numpy as jnp
from jax.experimental import pallas as pl


def kernel(x, W, sim_matrix, temperature):
    raise NotImplementedError("write your pallas kernel here")


# speedup vs baseline: 1.1063x; 1.1063x over previous
import jax
import jax.numpy as jnp
from jax.experimental import pallas as pl
from jax.experimental.pallas import tpu as pltpu

D_MODEL = 4096
NUM_EXPERTS = 64
TOP_K = 8

TI = 256
NJ = 4
DJ = D_MODEL // NJ
N_TOKENS = 4096
NI = N_TOKENS // TI

_CONTRACT_LAST = (((1,), (1,)), ((), ()))


def _router_kernel(temp_ref, x_ref, w_ref, sim_ref,
                   scores_ref, probs_ref, usage_ref, mean_ref,
                   s_acc, n2_acc, simn_acc, mean_acc):
    j = pl.program_id(0)
    i = pl.program_id(1)

    p = jax.lax.dot_general(x_ref[...], w_ref[...], _CONTRACT_LAST,
                            preferred_element_type=jnp.float32)
    n2_part = jnp.sum(p * p, axis=1, keepdims=True)
    s_part = jax.lax.dot_general(p, sim_ref[...], _CONTRACT_LAST,
                                 preferred_element_type=jnp.float32)

    rows = pl.ds(i * TI, TI)

    @pl.when(j == 0)
    def _init():
        s_acc[rows, :] = s_part
        n2_acc[rows, :] = n2_part

    @pl.when(j > 0)
    def _accum():
        s_acc[rows, :] = s_acc[rows, :] + s_part
        n2_acc[rows, :] = n2_acc[rows, :] + n2_part

    @pl.when(i == 0)
    def _simnorm():
        ssq = sim_ref[...] * sim_ref[...]
        part = jax.lax.dot_general(
            jnp.ones((1, DJ), jnp.float32), ssq, _CONTRACT_LAST,
            preferred_element_type=jnp.float32)

        @pl.when(j == 0)
        def _():
            simn_acc[...] = part

        @pl.when(j > 0)
        def _():
            simn_acc[...] = simn_acc[...] + part

        @pl.when(j == NJ - 1)
        def _():
            temp = temp_ref[0, 0]
            simn_acc[...] = 1.0 / (
                jnp.maximum(jnp.sqrt(simn_acc[...]), 1e-12) * temp)

    @pl.when(j == NJ - 1)
    def _finalize():
        s = s_acc[rows, :]
        pnorm = jnp.maximum(jnp.sqrt(n2_acc[rows, :]), 1e-12)
        scores = (s / pnorm) * simn_acc[...]
        scores_ref[...] = scores

        lane = jax.lax.broadcasted_iota(jnp.int32, (TI, NUM_EXPERTS), 1)
        work = scores
        sel_total = jnp.zeros((TI, NUM_EXPERTS), jnp.float32)
        vals = []
        for _ in range(TOP_K):
            m = jnp.max(work, axis=1, keepdims=True)
            ismax = work == m
            midx = jnp.min(jnp.where(ismax, lane, NUM_EXPERTS),
                           axis=1, keepdims=True)
            sel = lane == midx
            vals.append(m)
            sel_total = sel_total + sel.astype(jnp.float32)
            work = jnp.where(sel, -jnp.inf, work)
        topv = jnp.concatenate(vals, axis=1)

        ex = jnp.exp(topv - vals[0])
        probs = ex / jnp.sum(ex, axis=1, keepdims=True)
        probs_ref[...] = probs

        usage_part = jnp.sum(sel_total, axis=0, keepdims=True)

        @pl.when(i == 0)
        def _():
            usage_ref[...] = usage_part
            mean_acc[0, 0] = jnp.sum(probs)

        @pl.when(i > 0)
        def _():
            usage_ref[...] = usage_ref[...] + usage_part
            mean_acc[0, 0] = mean_acc[0, 0] + jnp.sum(probs)

        @pl.when(i == NI - 1)
        def _():
            mean_ref[...] = jnp.full(
                (1, 1), mean_acc[0, 0] * (1.0 / (N_TOKENS * TOP_K)),
                jnp.float32)


def kernel(x, W, sim_matrix, temperature):
    B, T, _ = x.shape
    x2d = x.reshape(N_TOKENS, D_MODEL)
    temp = jnp.asarray(temperature, jnp.float32).reshape(1, 1)

    grid = (NJ, NI)
    scores, probs, usage, mean = pl.pallas_call(
        _router_kernel,
        grid=grid,
        in_specs=[
            pl.BlockSpec(memory_space=pltpu.SMEM),
            pl.BlockSpec((TI, D_MODEL), lambda j, i: (i, 0)),
            pl.BlockSpec((DJ, D_MODEL), lambda j, i: (j, 0)),
            pl.BlockSpec((NUM_EXPERTS, DJ), lambda j, i: (0, j)),
        ],
        out_specs=[
            pl.BlockSpec((TI, NUM_EXPERTS), lambda j, i: (i, 0)),
            pl.BlockSpec((TI, TOP_K), lambda j, i: (i, 0)),
            pl.BlockSpec((1, NUM_EXPERTS), lambda j, i: (0, 0)),
            pl.BlockSpec((1, 1), lambda j, i: (0, 0)),
        ],
        out_shape=[
            jax.ShapeDtypeStruct((N_TOKENS, NUM_EXPERTS), jnp.float32),
            jax.ShapeDtypeStruct((N_TOKENS, TOP_K), jnp.float32),
            jax.ShapeDtypeStruct((1, NUM_EXPERTS), jnp.float32),
            jax.ShapeDtypeStruct((1, 1), jnp.float32),
        ],
        scratch_shapes=[
            pltpu.VMEM((N_TOKENS, NUM_EXPERTS), jnp.float32),
            pltpu.VMEM((N_TOKENS, 1), jnp.float32),
            pltpu.VMEM((1, NUM_EXPERTS), jnp.float32),
            pltpu.SMEM((1, 1), jnp.float32),
        ],
        compiler_params=pltpu.CompilerParams(
            dimension_semantics=("arbitrary", "arbitrary"),
        ),
    )(temp, x2d, W, sim_matrix)

    gate_scores = scores.reshape(B, T, NUM_EXPERTS)
    gate_probs = probs.reshape(B, T, TOP_K)
    expert_usage = usage.reshape(NUM_EXPERTS)
    gate_probs_mean = mean[0, 0]
    return gate_scores, gate_probs, expert_usage, gate_probs_mean


# speedup vs baseline: 1.2182x; 1.1012x over previous
import jax
import jax.numpy as jnp
from jax.experimental import pallas as pl
from jax.experimental.pallas import tpu as pltpu

D_MODEL = 4096
NUM_EXPERTS = 64
TOP_K = 8

TI = 256
NJ = 4
DJ = D_MODEL // NJ
N_TOKENS = 4096
NI = N_TOKENS // TI

_CONTRACT_LAST = (((1,), (1,)), ((), ()))


def _router_kernel(temp_ref, x_ref, w_ref, sim_ref,
                   scores_ref, probs_ref, usage_ref, mean_ref,
                   s_acc, n2_acc, simn_acc, mean_acc):
    j = pl.program_id(0)
    i = pl.program_id(1)

    p = jax.lax.dot_general(x_ref[...], w_ref[...], _CONTRACT_LAST,
                            preferred_element_type=jnp.float32)
    n2_part = jnp.sum(p * p, axis=1, keepdims=True)
    s_part = jax.lax.dot_general(p, sim_ref[...], _CONTRACT_LAST,
                                 preferred_element_type=jnp.float32)

    rows = pl.ds(i * TI, TI)

    @pl.when(j == 0)
    def _init():
        s_acc[rows, :] = s_part
        n2_acc[rows, :] = n2_part

    @pl.when(j > 0)
    def _accum():
        s_acc[rows, :] = s_acc[rows, :] + s_part
        n2_acc[rows, :] = n2_acc[rows, :] + n2_part

    @pl.when(i == 0)
    def _simnorm():
        ssq = sim_ref[...] * sim_ref[...]
        part = jax.lax.dot_general(
            jnp.ones((1, DJ), jnp.float32), ssq, _CONTRACT_LAST,
            preferred_element_type=jnp.float32)

        @pl.when(j == 0)
        def _():
            simn_acc[...] = part

        @pl.when(j > 0)
        def _():
            simn_acc[...] = simn_acc[...] + part

        @pl.when(j == NJ - 1)
        def _():
            temp = temp_ref[0, 0]
            simn_acc[...] = 1.0 / (
                jnp.maximum(jnp.sqrt(simn_acc[...]), 1e-12) * temp)

    @pl.when(j == NJ - 1)
    def _finalize():
        s = s_acc[rows, :]
        pnorm = jnp.maximum(jnp.sqrt(n2_acc[rows, :]), 1e-12)
        scores = (s / pnorm) * simn_acc[...]
        scores_ref[...] = scores

        st = scores.T
        eid = jax.lax.broadcasted_iota(jnp.int32, (NUM_EXPERTS, TI), 0)
        work = st
        sel_total = jnp.zeros((NUM_EXPERTS, TI), jnp.float32)
        vals = []
        for _ in range(TOP_K):
            m = jnp.max(work, axis=0, keepdims=True)
            ismax = work == m
            midx = jnp.min(jnp.where(ismax, eid, NUM_EXPERTS),
                           axis=0, keepdims=True)
            sel = eid == midx
            vals.append(m)
            sel_total = sel_total + sel.astype(jnp.float32)
            work = jnp.where(sel, -jnp.inf, work)
        topv = jnp.concatenate(vals, axis=0)

        ex = jnp.exp(topv - vals[0])
        probs_t = ex / jnp.sum(ex, axis=0, keepdims=True)
        probs_ref[...] = probs_t.T

        usage_part = jnp.sum(sel_total, axis=1, keepdims=True).T

        @pl.when(i == 0)
        def _():
            usage_ref[...] = usage_part
            mean_acc[0, 0] = jnp.sum(probs_t)

        @pl.when(i > 0)
        def _():
            usage_ref[...] = usage_ref[...] + usage_part
            mean_acc[0, 0] = mean_acc[0, 0] + jnp.sum(probs_t)

        @pl.when(i == NI - 1)
        def _():
            mean_ref[...] = jnp.full(
                (1, 1), mean_acc[0, 0] * (1.0 / (N_TOKENS * TOP_K)),
                jnp.float32)


def kernel(x, W, sim_matrix, temperature):
    B, T, _ = x.shape
    x2d = x.reshape(N_TOKENS, D_MODEL)
    temp = jnp.asarray(temperature, jnp.float32).reshape(1, 1)

    grid = (NJ, NI)
    scores, probs, usage, mean = pl.pallas_call(
        _router_kernel,
        grid=grid,
        in_specs=[
            pl.BlockSpec(memory_space=pltpu.SMEM),
            pl.BlockSpec((TI, D_MODEL), lambda j, i: (i, 0)),
            pl.BlockSpec((DJ, D_MODEL), lambda j, i: (j, 0)),
            pl.BlockSpec((NUM_EXPERTS, DJ), lambda j, i: (0, j)),
        ],
        out_specs=[
            pl.BlockSpec((TI, NUM_EXPERTS), lambda j, i: (i, 0)),
            pl.BlockSpec((TI, TOP_K), lambda j, i: (i, 0)),
            pl.BlockSpec((1, NUM_EXPERTS), lambda j, i: (0, 0)),
            pl.BlockSpec((1, 1), lambda j, i: (0, 0)),
        ],
        out_shape=[
            jax.ShapeDtypeStruct((N_TOKENS, NUM_EXPERTS), jnp.float32),
            jax.ShapeDtypeStruct((N_TOKENS, TOP_K), jnp.float32),
            jax.ShapeDtypeStruct((1, NUM_EXPERTS), jnp.float32),
            jax.ShapeDtypeStruct((1, 1), jnp.float32),
        ],
        scratch_shapes=[
            pltpu.VMEM((N_TOKENS, NUM_EXPERTS), jnp.float32),
            pltpu.VMEM((N_TOKENS, 1), jnp.float32),
            pltpu.VMEM((1, NUM_EXPERTS), jnp.float32),
            pltpu.SMEM((1, 1), jnp.float32),
        ],
        compiler_params=pltpu.CompilerParams(
            dimension_semantics=("arbitrary", "arbitrary"),
        ),
    )(temp, x2d, W, sim_matrix)

    gate_scores = scores.reshape(B, T, NUM_EXPERTS)
    gate_probs = probs.reshape(B, T, TOP_K)
    expert_usage = usage.reshape(NUM_EXPERTS)
    gate_probs_mean = mean[0, 0]
    return gate_scores, gate_probs, expert_usage, gate_probs_mean


# speedup vs baseline: 1.3534x; 1.1110x over previous
import jax
import jax.numpy as jnp
from jax.experimental import pallas as pl
from jax.experimental.pallas import tpu as pltpu

D_MODEL = 4096
NUM_EXPERTS = 64
TOP_K = 8

TI = 512
NJ = 4
DJ = D_MODEL // NJ
N_TOKENS = 4096
NI = N_TOKENS // TI

_CONTRACT_LAST = (((1,), (1,)), ((), ()))


def _router_kernel(temp_ref, x_ref, w_ref, sim_ref,
                   scores_ref, probs_ref, usage_ref, mean_ref,
                   s_acc, n2_acc, simn_acc, mean_acc):
    j = pl.program_id(0)
    i = pl.program_id(1)

    p = jax.lax.dot_general(x_ref[...], w_ref[...], _CONTRACT_LAST,
                            preferred_element_type=jnp.float32)
    n2_part = jnp.sum(p * p, axis=1, keepdims=True)
    s_part = jax.lax.dot_general(p, sim_ref[...], _CONTRACT_LAST,
                                 preferred_element_type=jnp.float32)

    rows = pl.ds(i * TI, TI)

    @pl.when(j == 0)
    def _init():
        s_acc[rows, :] = s_part
        n2_acc[rows, :] = n2_part

    @pl.when(j > 0)
    def _accum():
        s_acc[rows, :] = s_acc[rows, :] + s_part
        n2_acc[rows, :] = n2_acc[rows, :] + n2_part

    @pl.when(i == 0)
    def _simnorm():
        ssq = sim_ref[...] * sim_ref[...]
        part = jax.lax.dot_general(
            jnp.ones((1, DJ), jnp.float32), ssq, _CONTRACT_LAST,
            preferred_element_type=jnp.float32)

        @pl.when(j == 0)
        def _():
            simn_acc[...] = part

        @pl.when(j > 0)
        def _():
            simn_acc[...] = simn_acc[...] + part

        @pl.when(j == NJ - 1)
        def _():
            temp = temp_ref[0, 0]
            simn_acc[...] = 1.0 / (
                jnp.maximum(jnp.sqrt(simn_acc[...]), 1e-12) * temp)

    @pl.when(j == NJ - 1)
    def _finalize():
        s = s_acc[rows, :]
        pnorm = jnp.maximum(jnp.sqrt(n2_acc[rows, :]), 1e-12)
        scores = (s / pnorm) * simn_acc[...]
        scores_ref[...] = scores

        st = scores.T
        eid = jax.lax.broadcasted_iota(jnp.int32, (NUM_EXPERTS, TI), 0)
        work = st
        sel_total = jnp.zeros((NUM_EXPERTS, TI), jnp.float32)
        vals = []
        for _ in range(TOP_K):
            m = jnp.max(work, axis=0, keepdims=True)
            ismax = work == m
            midx = jnp.min(jnp.where(ismax, eid, NUM_EXPERTS),
                           axis=0, keepdims=True)
            sel = eid == midx
            vals.append(m)
            sel_total = sel_total + sel.astype(jnp.float32)
            work = jnp.where(sel, -jnp.inf, work)
        topv = jnp.concatenate(vals, axis=0)

        ex = jnp.exp(topv - vals[0])
        probs_t = ex / jnp.sum(ex, axis=0, keepdims=True)
        probs_ref[...] = probs_t.T

        usage_part = jnp.sum(sel_total, axis=1, keepdims=True).T

        @pl.when(i == 0)
        def _():
            usage_ref[...] = usage_part
            mean_acc[0, 0] = jnp.sum(probs_t)

        @pl.when(i > 0)
        def _():
            usage_ref[...] = usage_ref[...] + usage_part
            mean_acc[0, 0] = mean_acc[0, 0] + jnp.sum(probs_t)

        @pl.when(i == NI - 1)
        def _():
            mean_ref[...] = jnp.full(
                (1, 1), mean_acc[0, 0] * (1.0 / (N_TOKENS * TOP_K)),
                jnp.float32)


def kernel(x, W, sim_matrix, temperature):
    B, T, _ = x.shape
    x2d = x.reshape(N_TOKENS, D_MODEL)
    temp = jnp.asarray(temperature, jnp.float32).reshape(1, 1)

    grid = (NJ, NI)
    scores, probs, usage, mean = pl.pallas_call(
        _router_kernel,
        grid=grid,
        in_specs=[
            pl.BlockSpec(memory_space=pltpu.SMEM),
            pl.BlockSpec((TI, D_MODEL), lambda j, i: (i, 0)),
            pl.BlockSpec((DJ, D_MODEL), lambda j, i: (j, 0)),
            pl.BlockSpec((NUM_EXPERTS, DJ), lambda j, i: (0, j)),
        ],
        out_specs=[
            pl.BlockSpec((TI, NUM_EXPERTS), lambda j, i: (i, 0)),
            pl.BlockSpec((TI, TOP_K), lambda j, i: (i, 0)),
            pl.BlockSpec((1, NUM_EXPERTS), lambda j, i: (0, 0)),
            pl.BlockSpec((1, 1), lambda j, i: (0, 0)),
        ],
        out_shape=[
            jax.ShapeDtypeStruct((N_TOKENS, NUM_EXPERTS), jnp.float32),
            jax.ShapeDtypeStruct((N_TOKENS, TOP_K), jnp.float32),
            jax.ShapeDtypeStruct((1, NUM_EXPERTS), jnp.float32),
            jax.ShapeDtypeStruct((1, 1), jnp.float32),
        ],
        scratch_shapes=[
            pltpu.VMEM((N_TOKENS, NUM_EXPERTS), jnp.float32),
            pltpu.VMEM((N_TOKENS, 1), jnp.float32),
            pltpu.VMEM((1, NUM_EXPERTS), jnp.float32),
            pltpu.SMEM((1, 1), jnp.float32),
        ],
        compiler_params=pltpu.CompilerParams(
            dimension_semantics=("arbitrary", "arbitrary"),
        ),
    )(temp, x2d, W, sim_matrix)

    gate_scores = scores.reshape(B, T, NUM_EXPERTS)
    gate_probs = probs.reshape(B, T, TOP_K)
    expert_usage = usage.reshape(NUM_EXPERTS)
    gate_probs_mean = mean[0, 0]
    return gate_scores, gate_probs, expert_usage, gate_probs_mean
